# baseline (speedup 1.0000x reference)
import functools

import jax
import jax.numpy as jnp
from jax import lax
from jax.experimental import pallas as pl
from jax.experimental.pallas import tpu as pltpu
from jax.experimental.pallas import tpu_sc as plsc

D_MODEL = 64
BATCH = 1024
SEQ = 200
MAX_LENGTH = 1024

NUM_WORKERS = 32
CHUNK = 100
N_CHUNKS = BATCH * SEQ // CHUNK
CHUNKS_PER_W = N_CHUNKS // NUM_WORKERS
LANES = 16
SLICES_PER_ROW = D_MODEL // LANES


def _positional_encoding(length, depth):
    depth = depth / 2
    positions = jnp.arange(length, dtype=jnp.float32)[:, None]
    depths = jnp.arange(depth, dtype=jnp.float32)[None, :] / depth
    angle_rates = 1.0 / jnp.power(10000.0, depths)
    angle_rads = positions * angle_rates
    pos = jnp.concatenate([jnp.sin(angle_rads), jnp.cos(angle_rads)], axis=-1)
    return pos.astype(jnp.float32)


_MESH = plsc.VectorSubcoreMesh(core_axis_name="c", subcore_axis_name="s")


@functools.partial(
    pl.kernel,
    mesh=_MESH,
    compiler_params=pltpu.CompilerParams(use_tc_tiling_on_sc=False),
    out_type=jax.ShapeDtypeStruct((N_CHUNKS, CHUNK, D_MODEL), jnp.float32),
    scratch_types=[
        pltpu.VMEM((CHUNKS_PER_W, CHUNK), jnp.int32),
        pltpu.VMEM((2, CHUNK, D_MODEL), jnp.float32),
        pltpu.VMEM((CHUNK, D_MODEL), jnp.float32),
        pltpu.SemaphoreType.DMA,
    ],
)
def _emb_kernel(x_hbm, pos_hbm, table_hbm, out_hbm, idx_v, pos_v, rows_v, sem):
    wid = lax.axis_index("s") * 2 + lax.axis_index("c")
    base = wid * CHUNKS_PER_W
    pltpu.sync_copy(x_hbm.at[pl.ds(base, CHUNKS_PER_W)], idx_v)
    pltpu.sync_copy(pos_hbm, pos_v)

    def chunk_body(j, carry):
        pltpu.async_copy(table_hbm.at[idx_v.at[j]], rows_v, sem).wait()
        pj = lax.rem(j, 2)

        def row_body(r, c):
            for d in range(SLICES_PER_ROW):
                sl = pl.ds(d * LANES, LANES)
                rows_v[r, sl] = rows_v[r, sl] * 8.0 + pos_v[pj, r, sl]
            return c

        lax.fori_loop(0, CHUNK, row_body, 0)
        pltpu.sync_copy(rows_v, out_hbm.at[base + j])
        return carry

    lax.fori_loop(0, CHUNKS_PER_W, chunk_body, 0)


def kernel(x, table):
    pos = _positional_encoding(MAX_LENGTH, D_MODEL)[:SEQ].reshape(
        2, CHUNK, D_MODEL)
    x2 = x.reshape(N_CHUNKS, CHUNK).astype(jnp.int32)
    out = _emb_kernel(x2, pos, table)
    return out.reshape(BATCH, SEQ, D_MODEL)


# speedup vs baseline: 1.2177x; 1.2177x over previous
import functools

import jax
import jax.numpy as jnp
from jax import lax
from jax.experimental import pallas as pl
from jax.experimental.pallas import tpu as pltpu
from jax.experimental.pallas import tpu_sc as plsc

D_MODEL = 64
BATCH = 1024
SEQ = 200
MAX_LENGTH = 1024

NUM_WORKERS = 32
HALF = SEQ // 2
ROWS_PER_W = BATCH // NUM_WORKERS
NBUF = 4
N_T = ROWS_PER_W // NBUF
LANES = 16
SLICES = D_MODEL // LANES
R_UNROLL = 2


def _positional_encoding(length, depth):
    depth = depth / 2
    positions = jnp.arange(length, dtype=jnp.float32)[:, None]
    depths = jnp.arange(depth, dtype=jnp.float32)[None, :] / depth
    angle_rates = 1.0 / jnp.power(10000.0, depths)
    angle_rads = positions * angle_rates
    pos = jnp.concatenate([jnp.sin(angle_rads), jnp.cos(angle_rads)], axis=-1)
    return pos.astype(jnp.float32)


_MESH = plsc.VectorSubcoreMesh(core_axis_name="c", subcore_axis_name="s")


@functools.partial(
    pl.kernel,
    mesh=_MESH,
    compiler_params=pltpu.CompilerParams(use_tc_tiling_on_sc=False),
    out_type=jax.ShapeDtypeStruct((BATCH, SEQ, D_MODEL), jnp.float32),
    scratch_types=[
        pltpu.VMEM((2 * ROWS_PER_W, HALF), jnp.int32),
        pltpu.VMEM((SEQ, D_MODEL), jnp.float32),
        pltpu.VMEM((NBUF, SEQ, D_MODEL), jnp.float32),
        pltpu.SemaphoreType.DMA,
        pltpu.SemaphoreType.DMA,
    ],
)
def _emb_kernel(x_hbm, pos_hbm, table_hbm, out_hbm, idx_v, pos_v, rows_v,
                gsem, osem):
    wid = lax.axis_index("s") * 2 + lax.axis_index("c")
    row_base = wid * ROWS_PER_W
    pltpu.sync_copy(x_hbm.at[pl.ds(2 * row_base, 2 * ROWS_PER_W)], idx_v)
    pltpu.sync_copy(pos_hbm, pos_v)

    def start_gather(p, slot):
        pltpu.async_copy(table_hbm.at[idx_v.at[2 * p]],
                         rows_v.at[slot, pl.ds(0, HALF)], gsem)
        pltpu.async_copy(table_hbm.at[idx_v.at[2 * p + 1]],
                         rows_v.at[slot, pl.ds(HALF, HALF)], gsem)

    def wait_gather(slot):
        pltpu.make_async_copy(out_hbm.at[0], rows_v.at[slot], gsem).wait()

    def start_out(p, slot):
        pltpu.async_copy(rows_v.at[slot], out_hbm.at[row_base + p], osem)

    def wait_out(slot):
        pltpu.make_async_copy(rows_v.at[slot], out_hbm.at[0], osem).wait()

    start_gather(0, 0)

    def t_body(t, carry):
        for b in range(NBUF):
            p = NBUF * t + b
            nslot = (b + 1) % NBUF
            if b == NBUF - 1:
                @pl.when(t < N_T - 1)
                def _():
                    wait_out(nslot)
                    start_gather(p + 1, nslot)
            else:
                @pl.when(t >= 1)
                def _():
                    wait_out(nslot)
                start_gather(p + 1, nslot)
            wait_gather(b)

            def r_body(r, c):
                for dr in range(R_UNROLL):
                    for d in range(SLICES):
                        sl = pl.ds(d * LANES, LANES)
                        rr = R_UNROLL * r + dr
                        rows_v[b, rr, sl] = (rows_v[b, rr, sl] * 8.0
                                             + pos_v[rr, sl])
                return c

            lax.fori_loop(0, SEQ // R_UNROLL, r_body, 0)
            start_out(p, b)
        return carry

    lax.fori_loop(0, N_T, t_body, 0)
    for b in range(NBUF):
        wait_out(b)


def kernel(x, table):
    pos = _positional_encoding(MAX_LENGTH, D_MODEL)[:SEQ]
    x2 = x.reshape(2 * BATCH, HALF).astype(jnp.int32)
    return _emb_kernel(x2, pos, table)


# speedup vs baseline: 1.7987x; 1.4772x over previous
import functools

import jax
import jax.numpy as jnp
from jax import lax
from jax.experimental import pallas as pl
from jax.experimental.pallas import tpu as pltpu
from jax.experimental.pallas import tpu_sc as plsc

D_MODEL = 64
BATCH = 1024
SEQ = 200
MAX_LENGTH = 1024

NUM_WORKERS = 32
HALF = SEQ // 2
ROWS_PER_W = BATCH // NUM_WORKERS
NBUF = 4
N_T = ROWS_PER_W // NBUF
LANES = 16
SLICES = D_MODEL // LANES
R_UNROLL = 2


def _positional_encoding(length, depth):
    depth = depth / 2
    positions = jnp.arange(length, dtype=jnp.float32)[:, None]
    depths = jnp.arange(depth, dtype=jnp.float32)[None, :] / depth
    angle_rates = 1.0 / jnp.power(10000.0, depths)
    angle_rads = positions * angle_rates
    pos = jnp.concatenate([jnp.sin(angle_rads), jnp.cos(angle_rads)], axis=-1)
    return pos.astype(jnp.float32)


_MESH = plsc.VectorSubcoreMesh(core_axis_name="c", subcore_axis_name="s")


@functools.partial(
    pl.kernel,
    mesh=_MESH,
    compiler_params=pltpu.CompilerParams(use_tc_tiling_on_sc=False),
    out_type=jax.ShapeDtypeStruct((BATCH, SEQ, D_MODEL), jnp.float32),
    scratch_types=[
        pltpu.VMEM((2 * ROWS_PER_W, HALF), jnp.int32),
        pltpu.VMEM((SEQ, D_MODEL), jnp.float32),
        pltpu.VMEM((NBUF, SEQ, D_MODEL), jnp.float32),
        pltpu.SemaphoreType.DMA,
        pltpu.SemaphoreType.DMA,
    ],
)
def _emb_kernel(x_hbm, pos_hbm, table_hbm, out_hbm, idx_v, pos_v, rows_v,
                gsem, osem):
    wid = lax.axis_index("s") * 2 + lax.axis_index("c")
    row_base = wid * ROWS_PER_W
    pltpu.sync_copy(x_hbm.at[pl.ds(2 * row_base, 2 * ROWS_PER_W)], idx_v)
    pltpu.sync_copy(pos_hbm, pos_v)

    def start_gather(p, slot):
        pltpu.async_copy(table_hbm.at[idx_v.at[2 * p]],
                         rows_v.at[slot, pl.ds(0, HALF)], gsem)
        pltpu.async_copy(table_hbm.at[idx_v.at[2 * p + 1]],
                         rows_v.at[slot, pl.ds(HALF, HALF)], gsem)

    def wait_gather(slot):
        pltpu.make_async_copy(out_hbm.at[0], rows_v.at[slot], gsem).wait()

    def start_out(p, slot):
        pltpu.async_copy(rows_v.at[slot], out_hbm.at[row_base + p], osem)

    def wait_out(slot):
        pltpu.make_async_copy(rows_v.at[slot], out_hbm.at[0], osem).wait()

    start_gather(0, 0)

    def t_body(t, carry):
        for b in range(NBUF):
            p = NBUF * t + b
            nslot = (b + 1) % NBUF
            if b == NBUF - 1:
                @pl.when(t < N_T - 1)
                def _():
                    wait_out(nslot)
                    start_gather(p + 1, nslot)
            else:
                @pl.when(t >= 1)
                def _():
                    wait_out(nslot)
                start_gather(p + 1, nslot)
            wait_gather(b)

            def r_body(r, c):
                for dr in range(R_UNROLL):
                    for d in range(SLICES):
                        sl = pl.ds(d * LANES, LANES)
                        rr = R_UNROLL * r + dr
                        rows_v[b, rr, sl] = (rows_v[b, rr, sl] * 8.0
                                             + pos_v[rr, sl])
                return c

            lax.fori_loop(0, SEQ // R_UNROLL, r_body, 0)
            start_out(p, b)
        return carry

    lax.fori_loop(0, N_T, t_body, 0)
    for b in range(NBUF):
        wait_out(b)


def kernel(x, table):
    from jax._src.pjit import with_layout_constraint
    from jax._src.layout import Layout
    table = with_layout_constraint(table, Layout(major_to_minor=(0, 1)))
    pos = _positional_encoding(MAX_LENGTH, D_MODEL)[:SEQ]
    x2 = x.reshape(2 * BATCH, HALF).astype(jnp.int32)
    return _emb_kernel(x2, pos, table)
